# Initial kernel scaffold; baseline (speedup 1.0000x reference)
import jax
import jax.numpy as jnp
from jax.experimental import pallas as pl


def kernel(image, fxfycxcy, c2w, label):
    raise NotImplementedError("write your pallas kernel here")


# speedup vs baseline: 3.8735x; 3.8735x over previous
import functools

import jax
import jax.numpy as jnp
from jax import lax
from jax.experimental import pallas as pl
from jax.experimental.pallas import tpu as pltpu
from jax.experimental.pallas import tpu_sc as plsc

_B, _V, _C, _H, _W = 16, 8, 3, 256, 256
_NIN, _NTG = 4, 6
_ROW = _C * _H * _W
_CHUNK = 3072
_CPR = _ROW // _CHUNK
_GRP = 16
_NC, _NS = 2, 16
_NWORK = _NC * _NS
_IN_CH = _B * _NIN * _CPR
_TG_CH = _B * _NTG * _CPR
_IN_PW = _IN_CH // _NWORK
_TG_PW = _TG_CH // _NWORK
_IN_G = _IN_PW // _GRP
_TG_G = _TG_PW // _GRP


def _body(img, lbl, small, tbl_in, tbl_tg, rows_in, rows_tg,
          img_in, img_tg, lbl_in, lbl_tg, small_in, small_tg,
          idx_in_v, idx_tg_v, buf, rin_v, rtg_v, sbuf_in, sbuf_tg, sem):
    wid = lax.axis_index("s") * _NC + lax.axis_index("c")

    pltpu.sync_copy(tbl_in.at[wid], idx_in_v)
    pltpu.sync_copy(tbl_tg.at[wid], idx_tg_v)

    def stream(src, idx_v, n_groups, out, base_chunk):
        def grp(g, carry):
            idx_reg = idx_v[pl.ds(g * _GRP, _GRP)]
            pltpu.async_copy(src.at[idx_reg], buf, sem).wait()
            pltpu.sync_copy(buf, out.at[pl.ds(base_chunk + g * _GRP, _GRP)])
            return carry
        lax.fori_loop(0, n_groups, grp, 0)

    stream(img, idx_in_v, _IN_G, img_in, wid * _IN_PW)
    stream(img, idx_tg_v, _TG_G, img_tg, wid * _TG_PW)
    stream(lbl, idx_in_v, _IN_G, lbl_in, wid * _IN_PW)
    stream(lbl, idx_tg_v, _TG_G, lbl_tg, wid * _TG_PW)

    @pl.when(wid == 0)
    def _():
        pltpu.sync_copy(rows_in, rin_v)
        pltpu.async_copy(small.at[rin_v], sbuf_in, sem).wait()
        pltpu.sync_copy(sbuf_in, small_in)

    @pl.when(wid == 1)
    def _():
        pltpu.sync_copy(rows_tg, rtg_v)
        pltpu.async_copy(small.at[rtg_v], sbuf_tg, sem).wait()
        pltpu.sync_copy(sbuf_tg, small_tg)


_copy = pl.kernel(
    _body,
    out_type=(
        jax.ShapeDtypeStruct((_IN_CH, _CHUNK), jnp.float32),
        jax.ShapeDtypeStruct((_TG_CH, _CHUNK), jnp.float32),
        jax.ShapeDtypeStruct((_IN_CH, _CHUNK), jnp.float32),
        jax.ShapeDtypeStruct((_TG_CH, _CHUNK), jnp.float32),
        jax.ShapeDtypeStruct((_B * _NIN, 128), jnp.float32),
        jax.ShapeDtypeStruct((_B * _NTG, 128), jnp.float32),
    ),
    mesh=plsc.VectorSubcoreMesh(core_axis_name="c", subcore_axis_name="s"),
    scratch_types=[
        pltpu.VMEM((_IN_PW,), jnp.int32),
        pltpu.VMEM((_TG_PW,), jnp.int32),
        pltpu.VMEM((_GRP, _CHUNK), jnp.float32),
        pltpu.VMEM((_B * _NIN,), jnp.int32),
        pltpu.VMEM((_B * _NTG,), jnp.int32),
        pltpu.VMEM((_B * _NIN, 128), jnp.float32),
        pltpu.VMEM((_B * _NTG, 128), jnp.float32),
        pltpu.SemaphoreType.DMA,
    ],
)


def _index_tables():
    u = jax.random.uniform(jax.random.key(42), (_B, _V))
    perm = jnp.argsort(u, axis=1)[:, :_NTG].astype(jnp.int32)
    base = jnp.arange(_B, dtype=jnp.int32)[:, None] * _V
    rows_in = (base + jnp.arange(_NIN, dtype=jnp.int32)[None, :]).reshape(-1)
    rows_tg = (base + perm).reshape(-1)
    carange = jnp.arange(_CPR, dtype=jnp.int32)[None, :]
    tbl_in = (rows_in[:, None] * _CPR + carange).reshape(_NWORK, _IN_PW)
    tbl_tg = (rows_tg[:, None] * _CPR + carange).reshape(_NWORK, _TG_PW)
    return rows_in, rows_tg, tbl_in, tbl_tg


def kernel(image, fxfycxcy, c2w, label):
    rows_in, rows_tg, tbl_in, tbl_tg = _index_tables()
    img = image.reshape(_B * _V * _CPR, _CHUNK)
    lbl = label.reshape(_B * _V * _CPR, _CHUNK)
    small = jnp.pad(
        jnp.concatenate([fxfycxcy.reshape(_B * _V, 4),
                         c2w.reshape(_B * _V, 16)], axis=1),
        ((0, 0), (0, 108)))
    (img_in, img_tg, lbl_in, lbl_tg,
     small_in, small_tg) = _copy(img, lbl, small,
                                 tbl_in, tbl_tg, rows_in, rows_tg)
    return (
        img_in.reshape(_B, _NIN, _C, _H, _W),
        small_in[:, :4].reshape(_B, _NIN, 4),
        small_in[:, 4:20].reshape(_B, _NIN, 4, 4),
        lbl_in.reshape(_B, _NIN, _C, _H, _W),
        img_tg.reshape(_B, _NTG, _C, _H, _W),
        small_tg[:, :4].reshape(_B, _NTG, 4),
        small_tg[:, 4:20].reshape(_B, _NTG, 4, 4),
        lbl_tg.reshape(_B, _NTG, _C, _H, _W),
    )


# speedup vs baseline: 3.9744x; 1.0260x over previous
import functools

import jax
import jax.numpy as jnp
from jax import lax
from jax.experimental import pallas as pl
from jax.experimental.pallas import tpu as pltpu
from jax.experimental.pallas import tpu_sc as plsc

_B, _V, _C, _H, _W = 16, 8, 3, 256, 256
_NIN, _NTG = 4, 6
_ROW = _C * _H * _W
_CHUNK = 3072
_CPR = _ROW // _CHUNK
_GRP = 16
_NC, _NS = 2, 16
_NWORK = _NC * _NS
_IN_CH = _B * _NIN * _CPR
_TG_CH = _B * _NTG * _CPR
_IN_PW = _IN_CH // _NWORK
_TG_PW = _TG_CH // _NWORK
_IN_G = _IN_PW // _GRP
_TG_G = _TG_PW // _GRP


def _body(img, lbl, small, tbl_in, tbl_tg, rows_in, rows_tg,
          img_in, img_tg, lbl_in, lbl_tg, small_in, small_tg,
          idx_in_v, idx_tg_v, buf0, buf1, rin_v, rtg_v, sbuf_in, sbuf_tg,
          sem, isem0, isem1, osem0, osem1):
    wid = lax.axis_index("s") * _NC + lax.axis_index("c")

    pltpu.sync_copy(tbl_in.at[wid], idx_in_v)
    pltpu.sync_copy(tbl_tg.at[wid], idx_tg_v)

    def stream(src, idx_v, n_groups, out, base_chunk):
        def gather(g, b, isem):
            pltpu.async_copy(src.at[idx_v[pl.ds(g * _GRP, _GRP)]], b, isem)

        def wait_gather(b, isem):
            pltpu.make_async_copy(src.at[pl.ds(0, _GRP)], b, isem).wait()

        def write(g, b, osem):
            pltpu.async_copy(b, out.at[pl.ds(base_chunk + g * _GRP, _GRP)],
                             osem)

        def wait_write(b, osem):
            pltpu.make_async_copy(b, out.at[pl.ds(base_chunk, _GRP)],
                                  osem).wait()

        gather(0, buf0, isem0)
        gather(1, buf1, isem1)

        def pair(k, carry):
            g = 2 * k
            wait_gather(buf0, isem0)
            write(g, buf0, osem0)
            wait_gather(buf1, isem1)
            write(g + 1, buf1, osem1)

            @pl.when(g + 2 < n_groups)
            def _():
                wait_write(buf0, osem0)
                gather(g + 2, buf0, isem0)
                wait_write(buf1, osem1)
                gather(g + 3, buf1, isem1)
            return carry

        lax.fori_loop(0, n_groups // 2, pair, 0)
        wait_write(buf0, osem0)
        wait_write(buf1, osem1)

    stream(img, idx_in_v, _IN_G, img_in, wid * _IN_PW)
    stream(img, idx_tg_v, _TG_G, img_tg, wid * _TG_PW)
    stream(lbl, idx_in_v, _IN_G, lbl_in, wid * _IN_PW)
    stream(lbl, idx_tg_v, _TG_G, lbl_tg, wid * _TG_PW)

    @pl.when(wid == 0)
    def _():
        pltpu.sync_copy(rows_in, rin_v)
        pltpu.async_copy(small.at[rin_v], sbuf_in, sem).wait()
        pltpu.sync_copy(sbuf_in, small_in)

    @pl.when(wid == 1)
    def _():
        pltpu.sync_copy(rows_tg, rtg_v)
        pltpu.async_copy(small.at[rtg_v], sbuf_tg, sem).wait()
        pltpu.sync_copy(sbuf_tg, small_tg)


_copy = pl.kernel(
    _body,
    out_type=(
        jax.ShapeDtypeStruct((_IN_CH, _CHUNK), jnp.float32),
        jax.ShapeDtypeStruct((_TG_CH, _CHUNK), jnp.float32),
        jax.ShapeDtypeStruct((_IN_CH, _CHUNK), jnp.float32),
        jax.ShapeDtypeStruct((_TG_CH, _CHUNK), jnp.float32),
        jax.ShapeDtypeStruct((_B * _NIN, 128), jnp.float32),
        jax.ShapeDtypeStruct((_B * _NTG, 128), jnp.float32),
    ),
    mesh=plsc.VectorSubcoreMesh(core_axis_name="c", subcore_axis_name="s"),
    scratch_types=[
        pltpu.VMEM((_IN_PW,), jnp.int32),
        pltpu.VMEM((_TG_PW,), jnp.int32),
        pltpu.VMEM((_GRP, _CHUNK), jnp.float32),
        pltpu.VMEM((_GRP, _CHUNK), jnp.float32),
        pltpu.VMEM((_B * _NIN,), jnp.int32),
        pltpu.VMEM((_B * _NTG,), jnp.int32),
        pltpu.VMEM((_B * _NIN, 128), jnp.float32),
        pltpu.VMEM((_B * _NTG, 128), jnp.float32),
        pltpu.SemaphoreType.DMA,
        pltpu.SemaphoreType.DMA,
        pltpu.SemaphoreType.DMA,
        pltpu.SemaphoreType.DMA,
        pltpu.SemaphoreType.DMA,
    ],
)


def _index_tables():
    u = jax.random.uniform(jax.random.key(42), (_B, _V))
    perm = jnp.argsort(u, axis=1)[:, :_NTG].astype(jnp.int32)
    base = jnp.arange(_B, dtype=jnp.int32)[:, None] * _V
    rows_in = (base + jnp.arange(_NIN, dtype=jnp.int32)[None, :]).reshape(-1)
    rows_tg = (base + perm).reshape(-1)
    carange = jnp.arange(_CPR, dtype=jnp.int32)[None, :]
    tbl_in = (rows_in[:, None] * _CPR + carange).reshape(_NWORK, _IN_PW)
    tbl_tg = (rows_tg[:, None] * _CPR + carange).reshape(_NWORK, _TG_PW)
    return rows_in, rows_tg, tbl_in, tbl_tg


def kernel(image, fxfycxcy, c2w, label):
    rows_in, rows_tg, tbl_in, tbl_tg = _index_tables()
    img = image.reshape(_B * _V * _CPR, _CHUNK)
    lbl = label.reshape(_B * _V * _CPR, _CHUNK)
    small = jnp.pad(
        jnp.concatenate([fxfycxcy.reshape(_B * _V, 4),
                         c2w.reshape(_B * _V, 16)], axis=1),
        ((0, 0), (0, 108)))
    (img_in, img_tg, lbl_in, lbl_tg,
     small_in, small_tg) = _copy(img, lbl, small,
                                 tbl_in, tbl_tg, rows_in, rows_tg)
    return (
        img_in.reshape(_B, _NIN, _C, _H, _W),
        small_in[:, :4].reshape(_B, _NIN, 4),
        small_in[:, 4:20].reshape(_B, _NIN, 4, 4),
        lbl_in.reshape(_B, _NIN, _C, _H, _W),
        img_tg.reshape(_B, _NTG, _C, _H, _W),
        small_tg[:, :4].reshape(_B, _NTG, 4),
        small_tg[:, 4:20].reshape(_B, _NTG, 4, 4),
        lbl_tg.reshape(_B, _NTG, _C, _H, _W),
    )
